# baseline (speedup 1.0000x reference)
import jax
import jax.numpy as jnp
from jax.experimental import pallas as pl
from jax.experimental.pallas import tpu as pltpu

_TOKEN_OFF = 4000


def _gather_tile_kernel(S, TILE, ids_ref, times_ref, tab_ref, pe_ref, out_ref):
    b = pl.program_id(0)
    s = pl.program_id(1)
    base = b * S + s * TILE
    for mi in range(TILE):
        i0 = ids_ref[base + mi]
        i1 = times_ref[base + mi]
        out_ref[0, mi] = tab_ref[i0, 0] + tab_ref[i1, 0]
    out_ref[0] = out_ref[0] + pe_ref[...]


def kernel(x, time, fused_table, pe_scaled):
    B, S = x.shape
    v_pad, d_model = fused_table.shape

    ids = x.astype(jnp.int32).reshape(B * S)
    times = (time.astype(jnp.int32) + _TOKEN_OFF).reshape(B * S)
    tab3d = fused_table.reshape(v_pad, 1, d_model)
    pe = pe_scaled[:S]

    tile = 128
    while S % tile:
        tile //= 2
    grid = (B, S // tile)

    import functools
    body = functools.partial(_gather_tile_kernel, S, tile)

    m_total = B * S
    bytes_accessed = (2 * m_total * 4
                      + v_pad * d_model * 4
                      + S * d_model * 4
                      + m_total * d_model * 4)
    cost = pl.CostEstimate(flops=3 * m_total * d_model, transcendentals=0,
                           bytes_accessed=bytes_accessed)

    out = pl.pallas_call(
        body,
        out_shape=jax.ShapeDtypeStruct((B, S, d_model), jnp.float32),
        grid=grid,
        in_specs=[
            pl.BlockSpec(memory_space=pltpu.SMEM),
            pl.BlockSpec(memory_space=pltpu.SMEM),
            pl.BlockSpec((v_pad, 1, d_model), lambda b, s: (0, 0, 0)),
            pl.BlockSpec((tile, d_model), lambda b, s: (s, 0)),
        ],
        out_specs=pl.BlockSpec((1, tile, d_model), lambda b, s: (b, s, 0)),
        compiler_params=pltpu.CompilerParams(
            dimension_semantics=("parallel", "parallel")),
        cost_estimate=cost,
    )(ids, times, tab3d, pe)
    return out


# speedup vs baseline: 1.6548x; 1.6548x over previous
import jax
import jax.numpy as jnp
from jax.experimental import pallas as pl
from jax.experimental.pallas import tpu as pltpu

_TOKEN_OFF = 4000


def _gather_tile_kernel(TILE, ids_ref, times_ref, tab_ref, pe_ref, out_ref):
    base = pl.program_id(0) * TILE
    for mi in range(TILE):
        i0 = ids_ref[base + mi]
        i1 = times_ref[base + mi]
        out_ref[mi] = tab_ref[i0, 0] + tab_ref[i1, 0] + pe_ref[mi]


def kernel(x, time, fused_table, pe_scaled):
    B, S = x.shape
    v_pad, d_model = fused_table.shape

    ids = x.astype(jnp.int32).reshape(B * S)
    times = (time.astype(jnp.int32) + _TOKEN_OFF).reshape(B * S)
    tab3d = fused_table.reshape(v_pad, 1, d_model)
    pe = pe_scaled[:S]

    tile = 256
    while S % tile:
        tile //= 2
    n_s = S // tile
    grid = (B * n_s,)

    import functools
    body = functools.partial(_gather_tile_kernel, tile)

    m_total = B * S
    bytes_accessed = (2 * m_total * 4
                      + v_pad * d_model * 4
                      + S * d_model * 4
                      + m_total * d_model * 4)
    cost = pl.CostEstimate(flops=3 * m_total * d_model, transcendentals=0,
                           bytes_accessed=bytes_accessed)

    out = pl.pallas_call(
        body,
        out_shape=jax.ShapeDtypeStruct((B * S, d_model), jnp.float32),
        grid=grid,
        in_specs=[
            pl.BlockSpec(memory_space=pltpu.SMEM),
            pl.BlockSpec(memory_space=pltpu.SMEM),
            pl.BlockSpec((v_pad, 1, d_model), lambda i: (0, 0, 0)),
            pl.BlockSpec((tile, d_model), lambda i: (i % n_s, 0)),
        ],
        out_specs=pl.BlockSpec((tile, d_model), lambda i: (i, 0)),
        compiler_params=pltpu.CompilerParams(
            dimension_semantics=("parallel",)),
        cost_estimate=cost,
    )(ids, times, tab3d, pe)
    return out.reshape(B, S, d_model)
